# baseline (speedup 1.0000x reference)
import functools

import jax
import jax.numpy as jnp
from jax import lax
from jax.experimental import pallas as pl
from jax.experimental.pallas import tpu as pltpu
from jax.experimental.pallas import tpu_sc as plsc

VOCAB = 1000000
MAXLEN = 200
EMBED_DIM = 64
BATCH = 4096

NC = 2
NS = 16
NW = NC * NS
BB = BATCH // 128
TT = MAXLEN // 8
TOKENS = BATCH * MAXLEN
PER_W = TOKENS // NW
CHUNK = 128
N_CHUNKS = MAXLEN
LANES = 16
NBUF = 4
DR = EMBED_DIM // 8
PAD = 129


def _body(idx_hbm, table_hbm, pos_hbm, out_hbm, idxb_v, pos_v, *bufs):
    rows = bufs[:NBUF]
    outb = bufs[NBUF:2 * NBUF]
    gsems = bufs[2 * NBUF:3 * NBUF]
    osems = bufs[3 * NBUF:]
    wid = lax.axis_index("s") * NC + lax.axis_index("c")

    for tr in range(TT):
        pltpu.sync_copy(
            idx_hbm.at[pl.ds((tr * BB + wid) * (8 * CHUNK), 8 * CHUNK)],
            idxb_v.at[pl.ds(tr * (8 * CHUNK), 8 * CHUNK)])
    pltpu.sync_copy(pos_hbm, pos_v)

    riota = lax.iota(jnp.int32, LANES)

    def gather(i, s):
        off = pl.multiple_of(i * CHUNK, CHUNK)
        return pltpu.make_async_copy(
            table_hbm.at[idxb_v.at[pl.ds(off, CHUNK)]],
            rows[s],
            gsems[s])

    def out_copy(i, s):
        return pltpu.make_async_copy(
            outb[s].at[pl.ds(0, DR), pl.ds(0, 8), pl.ds(0, 128)],
            out_hbm.at[i, pl.ds(0, DR), wid],
            osems[s])

    for s in range(NBUF - 1):
        gather(s, s).start()

    def chunk_body(i0, carry):
        for sl in range(NBUF):
            i = i0 * NBUF + sl
            sp = (sl + NBUF - 1) % NBUF
            pf = i + NBUF - 1

            @pl.when(pf < N_CHUNKS)
            def _():
                gather(pf, sp).start()

            gather(i, sl).wait()

            @pl.when(i >= NBUF)
            def _():
                out_copy(i - NBUF, sl).wait()

            pv = [pos_v[i, pl.ds(c * LANES, LANES)]
                  for c in range(EMBED_DIM // LANES)]
            drv = [lax.shift_right_logical(riota + (c * LANES), 3)
                   for c in range(EMBED_DIM // LANES)]
            div = [lax.bitwise_and(riota + (c * LANES), 7)
                   for c in range(EMBED_DIM // LANES)]

            def add_body(jj, c2):
                b = jj * 2
                for r in range(2):
                    bvec = jnp.full((LANES,), b + r, jnp.int32)
                    for c in range(EMBED_DIM // LANES):
                        v = rows[sl][b + r, pl.ds(c * LANES, LANES)] + pv[c]
                        plsc.store_scatter(
                            outb[sl], [drv[c], div[c], bvec], v)
                return c2

            lax.fori_loop(0, CHUNK // 2, add_body, 0)
            out_copy(i, sl).start()
        return carry

    lax.fori_loop(0, N_CHUNKS // NBUF, chunk_body, 0)
    for s in range(NBUF):
        out_copy(N_CHUNKS - NBUF + s, s).wait()


def kernel(inputs, token_table, pos_table):
    idx1 = jnp.reshape(
        jnp.transpose(
            jnp.reshape(inputs.astype(jnp.int32), (BB, 128, TT, 8)),
            (2, 0, 3, 1)),
        (TOKENS,))
    mesh = plsc.VectorSubcoreMesh(core_axis_name="c", subcore_axis_name="s")
    fn = functools.partial(
        pl.kernel,
        mesh=mesh,
        compiler_params=pltpu.CompilerParams(use_tc_tiling_on_sc=False,
                                             needs_layout_passes=False),
        out_type=jax.ShapeDtypeStruct((MAXLEN, DR, BB, 8, 128), jnp.float32),
        scratch_types=[
            pltpu.VMEM((PER_W,), jnp.int32),
            pltpu.VMEM((MAXLEN, EMBED_DIM), jnp.float32),
        ]
        + [pltpu.VMEM((CHUNK, EMBED_DIM), jnp.float32)] * NBUF
        + [pltpu.VMEM((DR, 8, PAD), jnp.float32)] * NBUF
        + [pltpu.SemaphoreType.DMA] * (2 * NBUF),
    )(_body)
    out5 = fn(idx1, token_table, pos_table)
    return jnp.reshape(
        jnp.transpose(out5, (2, 4, 0, 1, 3)), (BATCH, MAXLEN, EMBED_DIM))


# speedup vs baseline: 1.0068x; 1.0068x over previous
import functools

import jax
import jax.numpy as jnp
from jax import lax
from jax.experimental import pallas as pl
from jax.experimental.pallas import tpu as pltpu
from jax.experimental.pallas import tpu_sc as plsc

VOCAB = 1000000
MAXLEN = 200
EMBED_DIM = 64
BATCH = 4096

NC = 2
NS = 16
NW = NC * NS
BB = BATCH // 128
TT = MAXLEN // 8
TOKENS = BATCH * MAXLEN
PER_W = TOKENS // NW
CHUNK = 128
N_CHUNKS = MAXLEN
LANES = 16
NBUF = 4
DR = EMBED_DIM // 8
PAD = 129


def _body(idx_hbm, table_hbm, pos_hbm, out_hbm, idxb_v, pos_v, *bufs):
    rows = bufs[:NBUF]
    outb = bufs[NBUF:2 * NBUF]
    gsems = bufs[2 * NBUF:3 * NBUF]
    osems = bufs[3 * NBUF:]
    wid = lax.axis_index("s") * NC + lax.axis_index("c")

    pltpu.sync_copy(
        idx_hbm.at[pl.ds(0, MAXLEN), pl.ds(wid * CHUNK, CHUNK)], idxb_v)
    pltpu.sync_copy(pos_hbm, pos_v)

    riota = lax.iota(jnp.int32, LANES)

    def gather(i, s):
        return pltpu.make_async_copy(
            table_hbm.at[idxb_v.at[i]],
            rows[s],
            gsems[s])

    def out_copy(i, s):
        return pltpu.make_async_copy(
            outb[s].at[pl.ds(0, DR), pl.ds(0, 8), pl.ds(0, 128)],
            out_hbm.at[i, pl.ds(0, DR), wid],
            osems[s])

    for s in range(NBUF - 1):
        gather(s, s).start()

    def chunk_body(i0, carry):
        for sl in range(NBUF):
            i = i0 * NBUF + sl
            sp = (sl + NBUF - 1) % NBUF
            pf = i + NBUF - 1

            @pl.when(pf < N_CHUNKS)
            def _():
                gather(pf, sp).start()

            gather(i, sl).wait()

            @pl.when(i >= NBUF)
            def _():
                out_copy(i - NBUF, sl).wait()

            pv = [pos_v[i, pl.ds(c * LANES, LANES)]
                  for c in range(EMBED_DIM // LANES)]
            drv = [lax.shift_right_logical(riota + (c * LANES), 3)
                   for c in range(EMBED_DIM // LANES)]
            div = [lax.bitwise_and(riota + (c * LANES), 7)
                   for c in range(EMBED_DIM // LANES)]

            def add_body(jj, c2):
                b = jj * 2
                for r in range(2):
                    bvec = jnp.full((LANES,), b + r, jnp.int32)
                    for c in range(EMBED_DIM // LANES):
                        v = rows[sl][b + r, pl.ds(c * LANES, LANES)] + pv[c]
                        plsc.store_scatter(
                            outb[sl], [drv[c], div[c], bvec], v)
                return c2

            lax.fori_loop(0, CHUNK // 2, add_body, 0)
            out_copy(i, sl).start()
        return carry

    lax.fori_loop(0, N_CHUNKS // NBUF, chunk_body, 0)
    for s in range(NBUF):
        out_copy(N_CHUNKS - NBUF + s, s).wait()


def kernel(inputs, token_table, pos_table):
    idx1 = jnp.transpose(inputs.astype(jnp.int32))
    mesh = plsc.VectorSubcoreMesh(core_axis_name="c", subcore_axis_name="s")
    fn = functools.partial(
        pl.kernel,
        mesh=mesh,
        compiler_params=pltpu.CompilerParams(use_tc_tiling_on_sc=False,
                                             needs_layout_passes=False),
        out_type=jax.ShapeDtypeStruct((MAXLEN, DR, BB, 8, 128), jnp.float32),
        scratch_types=[
            pltpu.VMEM((MAXLEN, CHUNK), jnp.int32),
            pltpu.VMEM((MAXLEN, EMBED_DIM), jnp.float32),
        ]
        + [pltpu.VMEM((CHUNK, EMBED_DIM), jnp.float32)] * NBUF
        + [pltpu.VMEM((DR, 8, PAD), jnp.float32)] * NBUF
        + [pltpu.SemaphoreType.DMA] * (2 * NBUF),
    )(_body)
    out5 = fn(idx1, token_table, pos_table)
    return jnp.reshape(
        jnp.transpose(out5, (2, 4, 0, 1, 3)), (BATCH, MAXLEN, EMBED_DIM))
